# baseline (speedup 1.0000x reference)
import functools

import jax
import jax.numpy as jnp
from jax import lax
from jax.experimental import pallas as pl
from jax.experimental.pallas import tpu as pltpu
from jax.experimental.pallas import tpu_sc as plsc

_EPS = 1e-12


def _safe(den):
    return jnp.where(jnp.abs(den) < _EPS, _EPS, den)


def _tile_kernel(pts_ref, trisT_ref, fidx_ref, cu_ref, cv_ref, cw_ref, *, F, TQ):
    b = pl.program_id(0)
    p = pts_ref[0]
    px = p[:, 0:1]
    py = p[:, 1:2]
    pz = p[:, 2:3]
    t = trisT_ref[0]
    ax = t[0:1]; ay = t[1:2]; az = t[2:3]
    bx = t[3:4]; by = t[4:5]; bz = t[5:6]
    cx = t[6:7]; cy = t[7:8]; cz = t[8:9]

    abx = bx - ax; aby = by - ay; abz = bz - az
    acx = cx - ax; acy = cy - ay; acz = cz - az

    apx = px - ax; apy = py - ay; apz = pz - az
    d1 = abx * apx + aby * apy + abz * apz
    d2 = acx * apx + acy * apy + acz * apz
    bpx = px - bx; bpy = py - by; bpz = pz - bz
    d3 = abx * bpx + aby * bpy + abz * bpz
    d4 = acx * bpx + acy * bpy + acz * bpz
    cpx = px - cx; cpy = py - cy; cpz = pz - cz
    d5 = abx * cpx + aby * cpy + abz * cpz
    d6 = acx * cpx + acy * cpy + acz * cpz

    va = d3 * d6 - d5 * d4
    vb = d5 * d2 - d1 * d6
    vc = d1 * d4 - d3 * d2
    v_ab = d1 / _safe(d1 - d3)
    w_ac = d2 / _safe(d2 - d6)
    w_bc = (d4 - d3) / _safe((d4 - d3) + (d5 - d6))
    denom = _safe(va + vb + vc)
    v_in = vb / denom
    w_in = vc / denom

    u = 1.0 - v_in - w_in; v = v_in; w = w_in
    on_bc = (va <= 0) & ((d4 - d3) >= 0) & ((d5 - d6) >= 0)
    u = jnp.where(on_bc, 0.0, u); v = jnp.where(on_bc, 1.0 - w_bc, v); w = jnp.where(on_bc, w_bc, w)
    on_ac = (vb <= 0) & (d2 >= 0) & (d6 <= 0)
    u = jnp.where(on_ac, 1.0 - w_ac, u); v = jnp.where(on_ac, 0.0, v); w = jnp.where(on_ac, w_ac, w)
    on_ab = (vc <= 0) & (d1 >= 0) & (d3 <= 0)
    u = jnp.where(on_ab, 1.0 - v_ab, u); v = jnp.where(on_ab, v_ab, v); w = jnp.where(on_ab, 0.0, w)
    at_c = (d6 >= 0) & (d5 <= d6)
    u = jnp.where(at_c, 0.0, u); v = jnp.where(at_c, 0.0, v); w = jnp.where(at_c, 1.0, w)
    at_b = (d3 >= 0) & (d4 <= d3)
    u = jnp.where(at_b, 0.0, u); v = jnp.where(at_b, 1.0, v); w = jnp.where(at_b, 0.0, w)
    at_a = (d1 <= 0) & (d2 <= 0)
    u = jnp.where(at_a, 1.0, u); v = jnp.where(at_a, 0.0, v); w = jnp.where(at_a, 0.0, w)

    clx = u * ax + v * bx + w * cx
    cly = u * ay + v * by + w * cy
    clz = u * az + v * bz + w * cz
    dist2 = (clx - px) ** 2 + (cly - py) ** 2 + (clz - pz) ** 2

    minv = jnp.min(dist2, axis=1, keepdims=True)
    fio = jax.lax.broadcasted_iota(jnp.int32, (TQ, F), 1)
    idx = jnp.min(jnp.where(dist2 == minv, fio, F), axis=1, keepdims=True)
    oh = (fio == idx).astype(jnp.float32)

    uw = jnp.sum(u * oh, axis=1, keepdims=True)
    vw = jnp.sum(v * oh, axis=1, keepdims=True)
    ww = jnp.sum(w * oh, axis=1, keepdims=True)
    cu_ref[0] = jnp.clip(uw, 0.0, 1.0)
    cv_ref[0] = jnp.clip(vw, 0.0, 1.0)
    cw_ref[0] = jnp.clip(ww, 0.0, 1.0)
    fidx_ref[0] = idx + b * F


def _sc_gather_kernel(tab_hbm, fidx_hbm, g_hbm, idx_v, rows_v, sem,
                      *, BPW, NCHUNK):
    nc = plsc.get_sparse_core_info().num_cores
    wid = lax.axis_index("s") * nc + lax.axis_index("c")
    base = wid * BPW

    for j in range(NCHUNK):
        pltpu.sync_copy(fidx_hbm.at[pl.ds(base + j * 128, 128)], idx_v.at[j])
    copies = [pltpu.async_copy(tab_hbm.at[idx_v.at[j]], rows_v.at[j], sem)
              for j in range(NCHUNK)]
    for cp in copies:
        cp.wait()
    for j in range(NCHUNK):
        pltpu.sync_copy(rows_v.at[j], g_hbm.at[pl.ds(base + j * 128, 128)])


def _interp_kernel(g_ref, pts_ref, cu_ref, cv_ref, cw_ref,
                   res_ref, nrm_ref, cmp_ref, idx_ref):
    g = g_ref[0]
    p = pts_ref[0]
    cu = cu_ref[0]
    cv = cv_ref[0]
    cw = cw_ref[0]
    feat = cu * g[:, 0:9] + cv * g[:, 9:18] + cw * g[:, 18:27]
    res_ref[0] = feat[:, 0:3] - p
    nrm_ref[0] = feat[:, 3:6]
    cmp_ref[0] = feat[:, 6:9]
    fid0 = g[:, 27:28]
    fid1 = g[:, 28:29]
    fid2 = g[:, 29:30]
    m0 = (cu >= cv) & (cu >= cw)
    m1 = jnp.logical_not(m0) & (cv >= cw)
    sel = jnp.where(m0, fid0, jnp.where(m1, fid1, fid2))
    idx_ref[0] = sel.astype(jnp.int32)


def kernel(triangles, points, normals, cmaps, faces):
    B, F = triangles.shape[0], triangles.shape[1]
    Q = points.shape[1]
    BQ = B * Q
    TQ = 512
    NQ = Q // TQ

    trisT = triangles.reshape(B, F, 9).transpose(0, 2, 1)

    fidx, cu, cv, cw = pl.pallas_call(
        functools.partial(_tile_kernel, F=F, TQ=TQ),
        grid=(B, NQ),
        in_specs=[
            pl.BlockSpec((1, TQ, 3), lambda b, qi: (b, qi, 0)),
            pl.BlockSpec((1, 9, F), lambda b, qi: (b, 0, 0)),
        ],
        out_specs=(
            pl.BlockSpec((1, TQ, 1), lambda b, qi: (b, qi, 0)),
            pl.BlockSpec((1, TQ, 1), lambda b, qi: (b, qi, 0)),
            pl.BlockSpec((1, TQ, 1), lambda b, qi: (b, qi, 0)),
            pl.BlockSpec((1, TQ, 1), lambda b, qi: (b, qi, 0)),
        ),
        out_shape=(
            jax.ShapeDtypeStruct((B, Q, 1), jnp.int32),
            jax.ShapeDtypeStruct((B, Q, 1), jnp.float32),
            jax.ShapeDtypeStruct((B, Q, 1), jnp.float32),
            jax.ShapeDtypeStruct((B, Q, 1), jnp.float32),
        ),
    )(points, trisT)

    v0 = jnp.concatenate([triangles[:, :, 0, :], normals[:, :, 0, :], cmaps[:, :, 0, :]], axis=-1)
    v1 = jnp.concatenate([triangles[:, :, 1, :], normals[:, :, 1, :], cmaps[:, :, 1, :]], axis=-1)
    v2 = jnp.concatenate([triangles[:, :, 2, :], normals[:, :, 2, :], cmaps[:, :, 2, :]], axis=-1)
    W = 128
    pad = jnp.zeros((B, F, W - 30), jnp.float32)
    tab = jnp.concatenate([v0, v1, v2, faces.astype(jnp.float32), pad], axis=-1)
    tab2 = tab.reshape(B * F, W)

    info = plsc.get_sparse_core_info()
    nw = info.num_cores * info.num_subcores
    BPW = BQ // nw
    NCHUNK = BPW // 128

    sc = pl.kernel(
        functools.partial(_sc_gather_kernel, BPW=BPW, NCHUNK=NCHUNK),
        mesh=plsc.VectorSubcoreMesh(core_axis_name="c", subcore_axis_name="s"),
        out_type=jax.ShapeDtypeStruct((BQ, W), jnp.float32),
        scratch_types=[
            pltpu.VMEM((NCHUNK, 128), jnp.int32),
            pltpu.VMEM((NCHUNK, 128, W), jnp.float32),
            pltpu.SemaphoreType.DMA,
        ],
    )
    g = sc(tab2, fidx.reshape(BQ))
    g = g.reshape(B, Q, W)

    res, nrm, cmp_, idx = pl.pallas_call(
        _interp_kernel,
        grid=(B, NQ),
        in_specs=[
            pl.BlockSpec((1, TQ, 128), lambda b, qi: (b, qi, 0)),
            pl.BlockSpec((1, TQ, 3), lambda b, qi: (b, qi, 0)),
            pl.BlockSpec((1, TQ, 1), lambda b, qi: (b, qi, 0)),
            pl.BlockSpec((1, TQ, 1), lambda b, qi: (b, qi, 0)),
            pl.BlockSpec((1, TQ, 1), lambda b, qi: (b, qi, 0)),
        ],
        out_specs=(
            pl.BlockSpec((1, TQ, 3), lambda b, qi: (b, qi, 0)),
            pl.BlockSpec((1, TQ, 3), lambda b, qi: (b, qi, 0)),
            pl.BlockSpec((1, TQ, 3), lambda b, qi: (b, qi, 0)),
            pl.BlockSpec((1, TQ, 1), lambda b, qi: (b, qi, 0)),
        ),
        out_shape=(
            jax.ShapeDtypeStruct((B, Q, 3), jnp.float32),
            jax.ShapeDtypeStruct((B, Q, 3), jnp.float32),
            jax.ShapeDtypeStruct((B, Q, 3), jnp.float32),
            jax.ShapeDtypeStruct((B, Q, 1), jnp.int32),
        ),
    )(g, points, cu, cv, cw)
    return res, nrm, cmp_, idx[:, :, 0]


# speedup vs baseline: 1.0661x; 1.0661x over previous
import functools

import jax
import jax.numpy as jnp
from jax.experimental import pallas as pl

_EPS = 1e-12


def _safe(den):
    return jnp.where(jnp.abs(den) < _EPS, _EPS, den)


def _tile_kernel(pts_ref, trisT_ref, thi_ref, tmid_ref, tlo_ref,
                 res_ref, nrm_ref, cmp_ref, idx_ref, *, F, TQ):
    p = pts_ref[0]
    px = p[:, 0:1]
    py = p[:, 1:2]
    pz = p[:, 2:3]
    t = trisT_ref[0]
    ax = t[0:1]; ay = t[1:2]; az = t[2:3]
    bx = t[3:4]; by = t[4:5]; bz = t[5:6]
    cx = t[6:7]; cy = t[7:8]; cz = t[8:9]

    abx = bx - ax; aby = by - ay; abz = bz - az
    acx = cx - ax; acy = cy - ay; acz = cz - az

    apx = px - ax; apy = py - ay; apz = pz - az
    d1 = abx * apx + aby * apy + abz * apz
    d2 = acx * apx + acy * apy + acz * apz
    bpx = px - bx; bpy = py - by; bpz = pz - bz
    d3 = abx * bpx + aby * bpy + abz * bpz
    d4 = acx * bpx + acy * bpy + acz * bpz
    cpx = px - cx; cpy = py - cy; cpz = pz - cz
    d5 = abx * cpx + aby * cpy + abz * cpz
    d6 = acx * cpx + acy * cpy + acz * cpz

    va = d3 * d6 - d5 * d4
    vb = d5 * d2 - d1 * d6
    vc = d1 * d4 - d3 * d2
    v_ab = d1 / _safe(d1 - d3)
    w_ac = d2 / _safe(d2 - d6)
    w_bc = (d4 - d3) / _safe((d4 - d3) + (d5 - d6))
    denom = _safe(va + vb + vc)
    v_in = vb / denom
    w_in = vc / denom

    u = 1.0 - v_in - w_in; v = v_in; w = w_in
    on_bc = (va <= 0) & ((d4 - d3) >= 0) & ((d5 - d6) >= 0)
    u = jnp.where(on_bc, 0.0, u); v = jnp.where(on_bc, 1.0 - w_bc, v); w = jnp.where(on_bc, w_bc, w)
    on_ac = (vb <= 0) & (d2 >= 0) & (d6 <= 0)
    u = jnp.where(on_ac, 1.0 - w_ac, u); v = jnp.where(on_ac, 0.0, v); w = jnp.where(on_ac, w_ac, w)
    on_ab = (vc <= 0) & (d1 >= 0) & (d3 <= 0)
    u = jnp.where(on_ab, 1.0 - v_ab, u); v = jnp.where(on_ab, v_ab, v); w = jnp.where(on_ab, 0.0, w)
    at_c = (d6 >= 0) & (d5 <= d6)
    u = jnp.where(at_c, 0.0, u); v = jnp.where(at_c, 0.0, v); w = jnp.where(at_c, 1.0, w)
    at_b = (d3 >= 0) & (d4 <= d3)
    u = jnp.where(at_b, 0.0, u); v = jnp.where(at_b, 1.0, v); w = jnp.where(at_b, 0.0, w)
    at_a = (d1 <= 0) & (d2 <= 0)
    u = jnp.where(at_a, 1.0, u); v = jnp.where(at_a, 0.0, v); w = jnp.where(at_a, 0.0, w)

    clx = u * ax + v * bx + w * cx
    cly = u * ay + v * by + w * cy
    clz = u * az + v * bz + w * cz
    dist2 = (clx - px) ** 2 + (cly - py) ** 2 + (clz - pz) ** 2

    minv = jnp.min(dist2, axis=1, keepdims=True)
    fio = jax.lax.broadcasted_iota(jnp.int32, (TQ, F), 1)
    idx = jnp.min(jnp.where(dist2 == minv, fio, F), axis=1, keepdims=True)
    oh = (fio == idx).astype(jnp.float32)

    uw = jnp.sum(u * oh, axis=1, keepdims=True)
    vw = jnp.sum(v * oh, axis=1, keepdims=True)
    ww = jnp.sum(w * oh, axis=1, keepdims=True)
    cu = jnp.clip(uw, 0.0, 1.0)
    cv = jnp.clip(vw, 0.0, 1.0)
    cw = jnp.clip(ww, 0.0, 1.0)

    ohb = oh.astype(jnp.bfloat16)
    g = (jnp.dot(ohb, thi_ref[0], preferred_element_type=jnp.float32)
         + jnp.dot(ohb, tmid_ref[0], preferred_element_type=jnp.float32)
         ) + jnp.dot(ohb, tlo_ref[0], preferred_element_type=jnp.float32)
    feat = cu * g[:, 0:9] + cv * g[:, 9:18] + cw * g[:, 18:27]
    res_ref[0] = feat[:, 0:3] - p
    nrm_ref[0] = feat[:, 3:6]
    cmp_ref[0] = feat[:, 6:9]

    fid0 = g[:, 27:28]
    fid1 = g[:, 28:29]
    fid2 = g[:, 29:30]
    m0 = (cu >= cv) & (cu >= cw)
    m1 = jnp.logical_not(m0) & (cv >= cw)
    sel = jnp.where(m0, fid0, jnp.where(m1, fid1, fid2))
    idx_ref[0] = (sel + 0.5).astype(jnp.int32)


def kernel(triangles, points, normals, cmaps, faces):
    B, F = triangles.shape[0], triangles.shape[1]
    Q = points.shape[1]
    TQ = 512
    NQ = Q // TQ

    trisT = triangles.reshape(B, F, 9).transpose(0, 2, 1)
    v0 = jnp.concatenate([triangles[:, :, 0, :], normals[:, :, 0, :], cmaps[:, :, 0, :]], axis=-1)
    v1 = jnp.concatenate([triangles[:, :, 1, :], normals[:, :, 1, :], cmaps[:, :, 1, :]], axis=-1)
    v2 = jnp.concatenate([triangles[:, :, 2, :], normals[:, :, 2, :], cmaps[:, :, 2, :]], axis=-1)
    tab = jnp.concatenate([v0, v1, v2, faces.astype(jnp.float32)], axis=-1)
    t_hi = tab.astype(jnp.bfloat16)
    r1 = tab - t_hi.astype(jnp.float32)
    t_mid = r1.astype(jnp.bfloat16)
    t_lo = (r1 - t_mid.astype(jnp.float32)).astype(jnp.bfloat16)

    res, nrm, cmp_, idx = pl.pallas_call(
        functools.partial(_tile_kernel, F=F, TQ=TQ),
        grid=(B, NQ),
        in_specs=[
            pl.BlockSpec((1, TQ, 3), lambda b, qi: (b, qi, 0)),
            pl.BlockSpec((1, 9, F), lambda b, qi: (b, 0, 0)),
            pl.BlockSpec((1, F, 30), lambda b, qi: (b, 0, 0)),
            pl.BlockSpec((1, F, 30), lambda b, qi: (b, 0, 0)),
            pl.BlockSpec((1, F, 30), lambda b, qi: (b, 0, 0)),
        ],
        out_specs=(
            pl.BlockSpec((1, TQ, 3), lambda b, qi: (b, qi, 0)),
            pl.BlockSpec((1, TQ, 3), lambda b, qi: (b, qi, 0)),
            pl.BlockSpec((1, TQ, 3), lambda b, qi: (b, qi, 0)),
            pl.BlockSpec((1, TQ, 1), lambda b, qi: (b, qi, 0)),
        ),
        out_shape=(
            jax.ShapeDtypeStruct((B, Q, 3), jnp.float32),
            jax.ShapeDtypeStruct((B, Q, 3), jnp.float32),
            jax.ShapeDtypeStruct((B, Q, 3), jnp.float32),
            jax.ShapeDtypeStruct((B, Q, 1), jnp.int32),
        ),
    )(points, trisT, t_hi, t_mid, t_lo)
    return res, nrm, cmp_, idx[:, :, 0]


# speedup vs baseline: 1.0924x; 1.0247x over previous
import functools

import jax
import jax.numpy as jnp
from jax.experimental import pallas as pl

_EPS = 1e-12


def _safe(den):
    return jnp.where(jnp.abs(den) < _EPS, _EPS, den)


def _tile_kernel(pts_ref, trisT_ref, tcat_ref,
                 res_ref, nrm_ref, cmp_ref, idx_ref, *, F, TQ):
    p = pts_ref[0]
    px = p[:, 0:1]
    py = p[:, 1:2]
    pz = p[:, 2:3]
    t = trisT_ref[0]
    ax = t[0:1]; ay = t[1:2]; az = t[2:3]
    bx = t[3:4]; by = t[4:5]; bz = t[5:6]
    cx = t[6:7]; cy = t[7:8]; cz = t[8:9]

    abx = bx - ax; aby = by - ay; abz = bz - az
    acx = cx - ax; acy = cy - ay; acz = cz - az

    apx = px - ax; apy = py - ay; apz = pz - az
    d1 = abx * apx + aby * apy + abz * apz
    d2 = acx * apx + acy * apy + acz * apz
    bpx = px - bx; bpy = py - by; bpz = pz - bz
    d3 = abx * bpx + aby * bpy + abz * bpz
    d4 = acx * bpx + acy * bpy + acz * bpz
    cpx = px - cx; cpy = py - cy; cpz = pz - cz
    d5 = abx * cpx + aby * cpy + abz * cpz
    d6 = acx * cpx + acy * cpy + acz * cpz

    va = d3 * d6 - d5 * d4
    vb = d5 * d2 - d1 * d6
    vc = d1 * d4 - d3 * d2
    v_ab = d1 / _safe(d1 - d3)
    w_ac = d2 / _safe(d2 - d6)
    w_bc = (d4 - d3) / _safe((d4 - d3) + (d5 - d6))
    denom = _safe(va + vb + vc)
    v_in = vb / denom
    w_in = vc / denom

    u = 1.0 - v_in - w_in; v = v_in; w = w_in
    on_bc = (va <= 0) & ((d4 - d3) >= 0) & ((d5 - d6) >= 0)
    u = jnp.where(on_bc, 0.0, u); v = jnp.where(on_bc, 1.0 - w_bc, v); w = jnp.where(on_bc, w_bc, w)
    on_ac = (vb <= 0) & (d2 >= 0) & (d6 <= 0)
    u = jnp.where(on_ac, 1.0 - w_ac, u); v = jnp.where(on_ac, 0.0, v); w = jnp.where(on_ac, w_ac, w)
    on_ab = (vc <= 0) & (d1 >= 0) & (d3 <= 0)
    u = jnp.where(on_ab, 1.0 - v_ab, u); v = jnp.where(on_ab, v_ab, v); w = jnp.where(on_ab, 0.0, w)
    at_c = (d6 >= 0) & (d5 <= d6)
    u = jnp.where(at_c, 0.0, u); v = jnp.where(at_c, 0.0, v); w = jnp.where(at_c, 1.0, w)
    at_b = (d3 >= 0) & (d4 <= d3)
    u = jnp.where(at_b, 0.0, u); v = jnp.where(at_b, 1.0, v); w = jnp.where(at_b, 0.0, w)
    at_a = (d1 <= 0) & (d2 <= 0)
    u = jnp.where(at_a, 1.0, u); v = jnp.where(at_a, 0.0, v); w = jnp.where(at_a, 0.0, w)

    clx = u * ax + v * bx + w * cx
    cly = u * ay + v * by + w * cy
    clz = u * az + v * bz + w * cz
    dist2 = (clx - px) ** 2 + (cly - py) ** 2 + (clz - pz) ** 2

    minv = jnp.min(dist2, axis=1, keepdims=True)
    fio = jax.lax.broadcasted_iota(jnp.int32, (TQ, F), 1)
    idx = jnp.min(jnp.where(dist2 == minv, fio, F), axis=1, keepdims=True)
    oh = (fio == idx).astype(jnp.float32)

    uw = jnp.sum(u * oh, axis=1, keepdims=True)
    vw = jnp.sum(v * oh, axis=1, keepdims=True)
    ww = jnp.sum(w * oh, axis=1, keepdims=True)
    cu = jnp.clip(uw, 0.0, 1.0)
    cv = jnp.clip(vw, 0.0, 1.0)
    cw = jnp.clip(ww, 0.0, 1.0)

    ohb = oh.astype(jnp.bfloat16)
    g3 = jnp.dot(ohb, tcat_ref[0], preferred_element_type=jnp.float32)
    g = (g3[:, 0:30] + g3[:, 30:60]) + g3[:, 60:90]
    feat = cu * g[:, 0:9] + cv * g[:, 9:18] + cw * g[:, 18:27]
    res_ref[0] = feat[:, 0:3] - p
    nrm_ref[0] = feat[:, 3:6]
    cmp_ref[0] = feat[:, 6:9]

    fid0 = g[:, 27:28]
    fid1 = g[:, 28:29]
    fid2 = g[:, 29:30]
    m0 = (cu >= cv) & (cu >= cw)
    m1 = jnp.logical_not(m0) & (cv >= cw)
    sel = jnp.where(m0, fid0, jnp.where(m1, fid1, fid2))
    idx_ref[0] = (sel + 0.5).astype(jnp.int32)


def kernel(triangles, points, normals, cmaps, faces):
    B, F = triangles.shape[0], triangles.shape[1]
    Q = points.shape[1]
    TQ = 512
    NQ = Q // TQ

    trisT = triangles.reshape(B, F, 9).transpose(0, 2, 1)
    v0 = jnp.concatenate([triangles[:, :, 0, :], normals[:, :, 0, :], cmaps[:, :, 0, :]], axis=-1)
    v1 = jnp.concatenate([triangles[:, :, 1, :], normals[:, :, 1, :], cmaps[:, :, 1, :]], axis=-1)
    v2 = jnp.concatenate([triangles[:, :, 2, :], normals[:, :, 2, :], cmaps[:, :, 2, :]], axis=-1)
    tab = jnp.concatenate([v0, v1, v2, faces.astype(jnp.float32)], axis=-1)
    t_hi = tab.astype(jnp.bfloat16)
    r1 = tab - t_hi.astype(jnp.float32)
    t_mid = r1.astype(jnp.bfloat16)
    t_lo = (r1 - t_mid.astype(jnp.float32)).astype(jnp.bfloat16)
    tcat = jnp.concatenate([t_hi, t_mid, t_lo], axis=-1)

    res, nrm, cmp_, idx = pl.pallas_call(
        functools.partial(_tile_kernel, F=F, TQ=TQ),
        grid=(B, NQ),
        in_specs=[
            pl.BlockSpec((1, TQ, 3), lambda b, qi: (b, qi, 0)),
            pl.BlockSpec((1, 9, F), lambda b, qi: (b, 0, 0)),
            pl.BlockSpec((1, F, 90), lambda b, qi: (b, 0, 0)),
        ],
        out_specs=(
            pl.BlockSpec((1, TQ, 3), lambda b, qi: (b, qi, 0)),
            pl.BlockSpec((1, TQ, 3), lambda b, qi: (b, qi, 0)),
            pl.BlockSpec((1, TQ, 3), lambda b, qi: (b, qi, 0)),
            pl.BlockSpec((1, TQ, 1), lambda b, qi: (b, qi, 0)),
        ),
        out_shape=(
            jax.ShapeDtypeStruct((B, Q, 3), jnp.float32),
            jax.ShapeDtypeStruct((B, Q, 3), jnp.float32),
            jax.ShapeDtypeStruct((B, Q, 3), jnp.float32),
            jax.ShapeDtypeStruct((B, Q, 1), jnp.int32),
        ),
    )(points, trisT, tcat)
    return res, nrm, cmp_, idx[:, :, 0]
